# baseline (speedup 1.0000x reference)
import functools

import jax
import jax.numpy as jnp
from jax import lax
from jax.experimental import pallas as pl
from jax.experimental.pallas import tpu as pltpu
from jax.experimental.pallas import tpu_sc as plsc

D = 128
B = 4096
QL = 20
AL = 50
AL2 = 52
NW = 32
IW = B // NW
QIC = 4
AIC = 2
QNC = IW // QIC
ANC = IW // AIC
NB = 4


def _sc_gather(idx_q, idx_a, table):
    mesh = plsc.VectorSubcoreMesh(core_axis_name="c", subcore_axis_name="s")

    @functools.partial(
        pl.kernel,
        mesh=mesh,
        out_type=[
            jax.ShapeDtypeStruct((B, QL, D), jnp.float32),
            jax.ShapeDtypeStruct((B, AL, D), jnp.float32),
        ],
        scratch_types=(
            [pltpu.VMEM((IW * QL,), jnp.int32),
             pltpu.VMEM((IW * AL2,), jnp.int32)]
            + [pltpu.VMEM((QIC * QL, D), jnp.float32) for _ in range(NB)]
            + [pltpu.VMEM((AIC * AL2, D), jnp.float32) for _ in range(NB)]
            + [pltpu.SemaphoreType.DMA for _ in range(4 * NB)]
        ),
    )
    def k(idx_q_hbm, idx_a_hbm, table_hbm, out_q_hbm, out_a_hbm,
          idx_q_v, idx_a_v, *scratch):
        qrows = scratch[:NB]
        arows = scratch[NB:2 * NB]
        sems = scratch[2 * NB:]
        qg, qs = sems[:NB], sems[NB:2 * NB]
        ag, as_ = sems[2 * NB:3 * NB], sems[3 * NB:]
        wid = lax.axis_index("s") * 2 + lax.axis_index("c")
        item0 = wid * IW

        pltpu.sync_copy(idx_q_hbm.at[pl.ds(wid * IW * QL, IW * QL)], idx_q_v)
        pltpu.sync_copy(idx_a_hbm.at[pl.ds(wid * IW * AL2, IW * AL2)],
                        idx_a_v)

        def run(idx_v, out_hbm, rows, gsem, ssem, nchunks, ipc, lpad, l):
            nidx = ipc * lpad

            def body(t, carry):
                descs = []
                for b in range(NB):
                    c = t * NB + b

                    @pl.when(t > 0)
                    def _():
                        for _j in range(ipc):
                            pltpu.make_async_copy(
                                rows[b].at[pl.ds(0, l)], out_hbm.at[0],
                                ssem[b]).wait()

                    descs.append(pltpu.async_copy(
                        table_hbm.at[idx_v.at[pl.ds(c * nidx, nidx)]],
                        rows[b], gsem[b]))
                for b in range(NB):
                    descs[b].wait()
                    c = t * NB + b
                    for j in range(ipc):
                        pltpu.async_copy(
                            rows[b].at[pl.ds(j * lpad, l)],
                            out_hbm.at[item0 + c * ipc + j], ssem[b])
                return carry

            lax.fori_loop(0, nchunks // NB, body, 0)
            for b in range(NB):
                for _j in range(ipc):
                    pltpu.make_async_copy(
                        rows[b].at[pl.ds(0, l)], out_hbm.at[0],
                        ssem[b]).wait()

        run(idx_q_v, out_q_hbm, qrows, qg, qs, QNC, QIC, QL, QL)
        run(idx_a_v, out_a_hbm, arows, ag, as_, ANC, AIC, AL2, AL)

    return k(idx_q, idx_a, table)


def _masks(iq, ia):
    def body(q_ref, a_ref, mq_ref, ma_ref):
        for ref, out in ((q_ref, mq_ref), (a_ref, ma_ref)):
            x = ref[...].astype(jnp.float32)
            m = jnp.max(x, axis=1, keepdims=True)
            out[...] = jnp.ceil(x / m)

    nb = 8
    bb = B // nb
    return pl.pallas_call(
        body,
        grid=(nb,),
        in_specs=[
            pl.BlockSpec((bb, QL), lambda i: (i, 0)),
            pl.BlockSpec((bb, AL), lambda i: (i, 0)),
        ],
        out_specs=[
            pl.BlockSpec((bb, QL), lambda i: (i, 0)),
            pl.BlockSpec((bb, AL), lambda i: (i, 0)),
        ],
        out_shape=[
            jax.ShapeDtypeStruct((B, QL), jnp.float32),
            jax.ShapeDtypeStruct((B, AL), jnp.float32),
        ],
    )(iq, ia)


def kernel(input_question, input_answer, embeddings):
    ia_pad = jnp.pad(input_answer, ((0, 0), (0, AL2 - AL)))
    eq, ea = _sc_gather(
        input_question.reshape(-1), ia_pad.reshape(-1), embeddings)
    mq, ma = _masks(input_question, input_answer)
    return eq, ea, mq, ma


# speedup vs baseline: 2.8731x; 2.8731x over previous
import functools

import jax
import jax.numpy as jnp
from jax import lax
from jax.experimental import pallas as pl
from jax.experimental.pallas import tpu as pltpu
from jax.experimental.pallas import tpu_sc as plsc

D = 128
B = 4096
QL = 20
AL = 50
NW = 32
IW = B // NW
QIC = 8
AIC = 4
QNC = IW // QIC
ANC = IW // AIC
NB = 4
BUFROWS = AIC * AL


def _splits(n):
    out = []
    while n > 0:
        s = min(n, 128)
        out.append(s)
        n -= s
    return out


def _sc_gather(idx_q, idx_a, table):
    mesh = plsc.VectorSubcoreMesh(core_axis_name="c", subcore_axis_name="s")

    @functools.partial(
        pl.kernel,
        mesh=mesh,
        out_type=[
            jax.ShapeDtypeStruct((B, QL, D), jnp.float32),
            jax.ShapeDtypeStruct((B, AL, D), jnp.float32),
        ],
        scratch_types=(
            [pltpu.VMEM((IW * QL,), jnp.int32),
             pltpu.VMEM((IW * AL,), jnp.int32)]
            + [pltpu.VMEM((BUFROWS, D), jnp.float32) for _ in range(NB)]
            + [pltpu.SemaphoreType.DMA for _ in range(2 * NB)]
        ),
    )
    def k(idx_q_hbm, idx_a_hbm, table_hbm, out_q_hbm, out_a_hbm,
          idx_q_v, idx_a_v, *scratch):
        rows = scratch[:NB]
        gsem = scratch[NB:2 * NB]
        ssem = scratch[2 * NB:]
        wid = lax.axis_index("s") * 2 + lax.axis_index("c")
        item0 = wid * IW

        pltpu.sync_copy(idx_q_hbm.at[pl.ds(wid * IW * QL, IW * QL)], idx_q_v)
        pltpu.sync_copy(idx_a_hbm.at[pl.ds(wid * IW * AL, IW * AL)], idx_a_v)

        def drain_stores(b, nidx):
            pltpu.make_async_copy(
                rows[b].at[pl.ds(0, nidx)], table_hbm.at[pl.ds(0, nidx)],
                ssem[b]).wait()

        def run(idx_v, out_hbm, nchunks, ipc, l):
            nidx = ipc * l
            subs = _splits(nidx)

            def body(t, carry):
                descs = []
                for b in range(NB):
                    c = t * NB + b

                    @pl.when(t > 0)
                    def _():
                        drain_stores(b, nidx)

                    off = 0
                    for s in subs:
                        descs.append(pltpu.async_copy(
                            table_hbm.at[
                                idx_v.at[pl.ds(c * nidx + off, s)]],
                            rows[b].at[pl.ds(off, s)], gsem[b]))
                        off += s
                i = 0
                for b in range(NB):
                    for _s in subs:
                        descs[i].wait()
                        i += 1
                    c = t * NB + b
                    for j in range(ipc):
                        pltpu.async_copy(
                            rows[b].at[pl.ds(j * l, l)],
                            out_hbm.at[item0 + c * ipc + j], ssem[b])
                return carry

            lax.fori_loop(0, nchunks // NB, body, 0)
            for b in range(NB):
                drain_stores(b, nidx)

        run(idx_q_v, out_q_hbm, QNC, QIC, QL)
        run(idx_a_v, out_a_hbm, ANC, AIC, AL)

    return k(idx_q, idx_a, table)


def _masks(iq, ia):
    def body(q_ref, a_ref, mq_ref, ma_ref):
        for ref, out in ((q_ref, mq_ref), (a_ref, ma_ref)):
            x = ref[...].astype(jnp.float32)
            m = jnp.max(x, axis=1, keepdims=True)
            out[...] = jnp.ceil(x / m)

    nb = 8
    bb = B // nb
    return pl.pallas_call(
        body,
        grid=(nb,),
        in_specs=[
            pl.BlockSpec((bb, QL), lambda i: (i, 0)),
            pl.BlockSpec((bb, AL), lambda i: (i, 0)),
        ],
        out_specs=[
            pl.BlockSpec((bb, QL), lambda i: (i, 0)),
            pl.BlockSpec((bb, AL), lambda i: (i, 0)),
        ],
        out_shape=[
            jax.ShapeDtypeStruct((B, QL), jnp.float32),
            jax.ShapeDtypeStruct((B, AL), jnp.float32),
        ],
    )(iq, ia)


def kernel(input_question, input_answer, embeddings):
    eq, ea = _sc_gather(
        input_question.reshape(-1), input_answer.reshape(-1), embeddings)
    mq, ma = _masks(input_question, input_answer)
    return eq, ea, mq, ma


# speedup vs baseline: 2.9150x; 1.0146x over previous
import functools

import jax
import jax.numpy as jnp
from jax import lax
from jax.experimental import pallas as pl
from jax.experimental.pallas import tpu as pltpu
from jax.experimental.pallas import tpu_sc as plsc

D = 128
B = 4096
QL = 20
AL = 50
NW = 32
IW = B // NW
QIC = 8
AIC = 4
QNC = IW // QIC
ANC = IW // AIC
NB = 4
BUFROWS = AIC * AL


def _splits(n):
    out = []
    while n > 0:
        s = min(n, 128)
        out.append(s)
        n -= s
    return out


def _sc_gather_one(idx, table, nchunks, ipc, l):
    mesh = plsc.VectorSubcoreMesh(core_axis_name="c", subcore_axis_name="s")
    nidx = ipc * l
    subs = _splits(nidx)

    @functools.partial(
        pl.kernel,
        mesh=mesh,
        out_type=jax.ShapeDtypeStruct((B, l, D), jnp.float32),
        scratch_types=(
            [pltpu.VMEM((IW * l,), jnp.int32)]
            + [pltpu.VMEM((nidx, D), jnp.float32) for _ in range(NB)]
            + [pltpu.SemaphoreType.DMA for _ in range(2 * NB)]
        ),
    )
    def k(idx_hbm, table_hbm, out_hbm, idx_v, *scratch):
        rows = scratch[:NB]
        gsem = scratch[NB:2 * NB]
        ssem = scratch[2 * NB:]
        wid = lax.axis_index("s") * 2 + lax.axis_index("c")
        item0 = wid * IW

        pltpu.sync_copy(idx_hbm.at[pl.ds(wid * IW * l, IW * l)], idx_v)

        def drain_stores(b):
            pltpu.make_async_copy(
                rows[b], table_hbm.at[pl.ds(0, nidx)], ssem[b]).wait()

        def body(t, carry):
            descs = []
            for b in range(NB):
                c = t * NB + b

                @pl.when(t > 0)
                def _():
                    drain_stores(b)

                off = 0
                for s in subs:
                    descs.append(pltpu.async_copy(
                        table_hbm.at[idx_v.at[pl.ds(c * nidx + off, s)]],
                        rows[b].at[pl.ds(off, s)], gsem[b]))
                    off += s
            i = 0
            for b in range(NB):
                for _s in subs:
                    descs[i].wait()
                    i += 1
                c = t * NB + b
                for j in range(ipc):
                    pltpu.async_copy(
                        rows[b].at[pl.ds(j * l, l)],
                        out_hbm.at[item0 + c * ipc + j], ssem[b])
            return carry

        lax.fori_loop(0, nchunks // NB, body, 0)
        for b in range(NB):
            drain_stores(b)

    return k(idx, table)


def _masks(iq, ia):
    def body(q_ref, a_ref, mq_ref, ma_ref):
        for ref, out in ((q_ref, mq_ref), (a_ref, ma_ref)):
            x = ref[...].astype(jnp.float32)
            m = jnp.max(x, axis=1, keepdims=True)
            out[...] = jnp.ceil(x / m)

    nb = 8
    bb = B // nb
    return pl.pallas_call(
        body,
        grid=(nb,),
        in_specs=[
            pl.BlockSpec((bb, QL), lambda i: (i, 0)),
            pl.BlockSpec((bb, AL), lambda i: (i, 0)),
        ],
        out_specs=[
            pl.BlockSpec((bb, QL), lambda i: (i, 0)),
            pl.BlockSpec((bb, AL), lambda i: (i, 0)),
        ],
        out_shape=[
            jax.ShapeDtypeStruct((B, QL), jnp.float32),
            jax.ShapeDtypeStruct((B, AL), jnp.float32),
        ],
    )(iq, ia)


def kernel(input_question, input_answer, embeddings):
    eq = _sc_gather_one(
        input_question.reshape(-1), embeddings, QNC, QIC, QL)
    ea = _sc_gather_one(
        input_answer.reshape(-1), embeddings, ANC, AIC, AL)
    mq, ma = _masks(input_question, input_answer)
    return eq, ea, mq, ma
